# baseline (speedup 1.0000x reference)
import dataclasses
import functools

import jax
import jax.numpy as jnp
from jax import lax
from jax.experimental import pallas as pl
from jax.experimental.pallas import tpu as pltpu
from jax.experimental.pallas import tpu_sc as plsc

EPS = 1e-12

NUM_SC_CORES = 2
NUM_SC_SUBCORES = 16
NUM_WORKERS = NUM_SC_CORES * NUM_SC_SUBCORES
LANES = 16

CHUNK = 32
NBUF = 2
TC_TW = 512


def _sc_gather_sum(word_table, pos_table, ids, pids):
    n = ids.shape[0]
    d = word_table.shape[1]
    per_w = n // NUM_WORKERS
    assert per_w % (NBUF * CHUNK) == 0 and per_w % 8 == 0 and d % LANES == 0

    mesh = plsc.VectorSubcoreMesh(core_axis_name="c", subcore_axis_name="s")
    cp = pltpu.CompilerParams()
    if "needs_layout_passes" in pltpu.CompilerParams.__dataclass_fields__:
        cp = dataclasses.replace(cp, needs_layout_passes=False)

    @functools.partial(
        pl.kernel,
        out_type=jax.ShapeDtypeStruct((n, d // 2), jnp.int32),
        mesh=mesh,
        compiler_params=cp,
        scratch_types=(
            [pltpu.VMEM((per_w,), jnp.int32)] * 2
            + [pltpu.VMEM((CHUNK, d), jnp.float32)] * (2 * NBUF)
            + [pltpu.VMEM((CHUNK, d // 2), jnp.int32)] * NBUF
            + [pltpu.SemaphoreType.DMA] * (3 * NBUF)
        ),
    )
    def sc_kernel(wt_hbm, pt_hbm, wid_hbm, pid_hbm, out_hbm,
                  widx_v, pidx_v, *rest):
        row_bufs = rest[:2 * NBUF]
        out_bufs = rest[2 * NBUF:3 * NBUF]
        sems = rest[3 * NBUF:]
        bufs = tuple(
            (row_bufs[2 * b], row_bufs[2 * b + 1], out_bufs[b],
             sems[3 * b], sems[3 * b + 1], sems[3 * b + 2])
            for b in range(NBUF))

        wid = lax.axis_index("s") * NUM_SC_CORES + lax.axis_index("c")
        base = wid * per_w
        pltpu.sync_copy(wid_hbm.at[pl.ds(base, per_w)], widx_v)
        pltpu.sync_copy(pid_hbm.at[pl.ds(base, per_w)], pidx_v)

        def issue_gather(off, b):
            wb, pb, _, gsw, gsp, _ = bufs[b]
            pltpu.async_copy(wt_hbm.at[widx_v.at[pl.ds(off, CHUNK)]], wb, gsw)
            pltpu.async_copy(pt_hbm.at[pidx_v.at[pl.ds(off, CHUNK)]], pb, gsp)

        def wait_gather(b):
            wb, pb, _, gsw, gsp, _ = bufs[b]
            pltpu.make_async_copy(wt_hbm.at[widx_v.at[pl.ds(0, CHUNK)]],
                                  wb, gsw).wait()
            pltpu.make_async_copy(pt_hbm.at[pidx_v.at[pl.ds(0, CHUNK)]],
                                  pb, gsp).wait()

        def add_rows(b):
            wb, pb, ob, _, _, _ = bufs[b]
            half = d // 2

            @pl.loop(0, CHUNK)
            def _(r):
                for c in range(d // (2 * LANES)):
                    sa = pl.ds(c * LANES, LANES)
                    sb = pl.ds(half + c * LANES, LANES)
                    s_a = wb[r, sa] + pb[r, sa]
                    s_b = wb[r, sb] + pb[r, sb]
                    packed = plsc.pack(s_a, s_b,
                                       format=plsc.PackFormat.INTERLEAVED)
                    ob.at[r, pl.ds(c * LANES, LANES)][...] = (
                        plsc.bitcast(packed, jnp.int32))

        def issue_store(off, b):
            _, _, ob, _, _, ss = bufs[b]
            pltpu.async_copy(ob, out_hbm.at[pl.ds(base + off, CHUNK)], ss)

        def wait_store(b):
            _, _, ob, _, _, ss = bufs[b]
            pltpu.make_async_copy(ob, out_hbm.at[pl.ds(base, CHUNK)],
                                  ss).wait()

        for b in range(NBUF):
            issue_gather(b * CHUNK, b)

        @pl.loop(0, per_w, step=NBUF * CHUNK)
        def _(off):
            for b in range(NBUF):
                wait_gather(b)
                add_rows(b)
                issue_store(off + b * CHUNK, b)

            @pl.when(off + NBUF * CHUNK < per_w)
            def _():
                for b in range(NBUF):
                    wait_store(b)
                    issue_gather(off + (NBUF + b) * CHUNK, b)

        for b in range(NBUF):
            wait_store(b)

    return sc_kernel(word_table, pos_table, ids, pids)


def _ln_body(x_ref, g_ref, b_ref, o_ref):
    w = x_ref[...]
    first = lax.bitcast_convert_type(w << 16, jnp.float32)
    second = lax.bitcast_convert_type(
        w & jnp.int32(-65536), jnp.float32)
    x = jnp.concatenate([first, second], axis=-1)
    mean = jnp.mean(x, axis=-1, keepdims=True)
    msq = jnp.mean(x * x, axis=-1, keepdims=True)
    var = msq - mean * mean
    o_ref[...] = ((x - mean) * lax.rsqrt(var + EPS)) * g_ref[...] + b_ref[...]


def _ln_body_aliased(acc_ref, x_ref, g_ref, b_ref, o_ref):
    del acc_ref
    _ln_body(x_ref, g_ref, b_ref, o_ref)


def _tc_layernorm_into(acc, rows, gamma, beta, row0, bs):
    n, dh = rows.shape
    d = 2 * dh
    block0 = row0 // TC_TW
    row_spec = pl.BlockSpec((TC_TW, dh), lambda i: (i, 0))
    vec_spec = pl.BlockSpec((1, d), lambda i: (0, 0))
    in_specs = [row_spec, vec_spec, vec_spec]
    operands = [rows, gamma.reshape(1, d), beta.reshape(1, d)]
    body = _ln_body
    aliases = {}
    if acc is not None:
        in_specs.insert(0, pl.BlockSpec(memory_space=pl.ANY))
        operands.insert(0, acc)
        aliases = {0: 0}
        body = _ln_body_aliased
    return pl.pallas_call(
        body,
        grid=(n // TC_TW,),
        in_specs=in_specs,
        out_specs=pl.BlockSpec((TC_TW, d), lambda i: (block0 + i, 0)),
        out_shape=jax.ShapeDtypeStruct((bs, d), jnp.float32),
        input_output_aliases=aliases,
        compiler_params=pltpu.CompilerParams(
            dimension_semantics=("parallel",)),
    )(*operands)


NCHUNK = 1


def kernel(input_ids, position_ids, word_table, pos_table, gamma, beta):
    b, s = input_ids.shape
    d = word_table.shape[1]
    bs = b * s
    ids = input_ids.reshape(-1)
    pids = position_ids.reshape(-1)

    nc = bs // NCHUNK
    summed = [
        _sc_gather_sum(word_table, pos_table,
                       lax.slice(ids, (k * nc,), ((k + 1) * nc,)),
                       lax.slice(pids, (k * nc,), ((k + 1) * nc,)))
        for k in range(NCHUNK)
    ]
    acc = None
    for k, rows_k in enumerate(summed):
        acc = _tc_layernorm_into(acc, rows_k, gamma, beta, k * nc, bs)
    return acc.reshape(b, s, d)


# speedup vs baseline: 1.4550x; 1.4550x over previous
import functools

import jax
import jax.numpy as jnp
from jax import lax
from jax.experimental import pallas as pl
from jax.experimental.pallas import tpu as pltpu
from jax.experimental.pallas import tpu_sc as plsc

EPS = 1e-12

NUM_SC_CORES = 2
NUM_SC_SUBCORES = 16
NUM_WORKERS = NUM_SC_CORES * NUM_SC_SUBCORES
LANES = 16

CHUNK = 32
NBUF = 2
TC_TW = 512


def _sc_gather_sum(word_table, pos_table, ids, pids):
    n = ids.shape[0]
    d = word_table.shape[1]
    per_w = n // NUM_WORKERS
    assert per_w % (NBUF * CHUNK) == 0 and per_w % 8 == 0 and d % LANES == 0

    mesh = plsc.VectorSubcoreMesh(core_axis_name="c", subcore_axis_name="s")

    @functools.partial(
        pl.kernel,
        out_type=jax.ShapeDtypeStruct((n, d), jnp.float32),
        mesh=mesh,
        scratch_types=(
            [pltpu.VMEM((per_w,), jnp.int32)] * 2
            + [pltpu.VMEM((CHUNK, d), jnp.float32)] * (2 * NBUF)
            + [pltpu.SemaphoreType.DMA] * (3 * NBUF)
        ),
    )
    def sc_kernel(wt_hbm, pt_hbm, wid_hbm, pid_hbm, out_hbm,
                  widx_v, pidx_v, *rest):
        row_bufs = rest[:2 * NBUF]
        sems = rest[2 * NBUF:]
        bufs = tuple(
            (row_bufs[2 * b], row_bufs[2 * b + 1],
             sems[3 * b], sems[3 * b + 1], sems[3 * b + 2])
            for b in range(NBUF))

        wid = lax.axis_index("s") * NUM_SC_CORES + lax.axis_index("c")
        base = wid * per_w
        pltpu.sync_copy(wid_hbm.at[pl.ds(base, per_w)], widx_v)
        pltpu.sync_copy(pid_hbm.at[pl.ds(base, per_w)], pidx_v)

        def issue_gather(off, b):
            wb, pb, gsw, gsp, _ = bufs[b]
            pltpu.async_copy(wt_hbm.at[widx_v.at[pl.ds(off, CHUNK)]], wb, gsw)
            pltpu.async_copy(pt_hbm.at[pidx_v.at[pl.ds(off, CHUNK)]], pb, gsp)

        def wait_gather(b):
            wb, pb, gsw, gsp, _ = bufs[b]
            pltpu.make_async_copy(wt_hbm.at[widx_v.at[pl.ds(0, CHUNK)]],
                                  wb, gsw).wait()
            pltpu.make_async_copy(pt_hbm.at[pidx_v.at[pl.ds(0, CHUNK)]],
                                  pb, gsp).wait()

        def add_rows(b):
            wb, pb, _, _, _ = bufs[b]

            @pl.loop(0, CHUNK)
            def _(r):
                for c in range(d // LANES):
                    sl = pl.ds(c * LANES, LANES)
                    plsc.addupdate(wb.at[r, sl], pb[r, sl])

        def issue_store(off, b):
            wb, _, _, _, ss = bufs[b]
            pltpu.async_copy(wb, out_hbm.at[pl.ds(base + off, CHUNK)], ss)

        def wait_store(b):
            wb, _, _, _, ss = bufs[b]
            pltpu.make_async_copy(wb, out_hbm.at[pl.ds(base, CHUNK)],
                                  ss).wait()

        for b in range(NBUF):
            issue_gather(b * CHUNK, b)

        @pl.loop(0, per_w, step=NBUF * CHUNK)
        def _(off):
            for b in range(NBUF):
                wait_gather(b)
                add_rows(b)
                issue_store(off + b * CHUNK, b)

            @pl.when(off + NBUF * CHUNK < per_w)
            def _():
                for b in range(NBUF):
                    wait_store(b)
                    issue_gather(off + (NBUF + b) * CHUNK, b)

        for b in range(NBUF):
            wait_store(b)

    return sc_kernel(word_table, pos_table, ids, pids)


def _ln_body(x_ref, g_ref, b_ref, o_ref):
    x = x_ref[...]
    mean = jnp.mean(x, axis=-1, keepdims=True)
    msq = jnp.mean(x * x, axis=-1, keepdims=True)
    var = msq - mean * mean
    o_ref[...] = ((x - mean) * lax.rsqrt(var + EPS)) * g_ref[...] + b_ref[...]


def _ln_body_aliased(acc_ref, x_ref, g_ref, b_ref, o_ref):
    del acc_ref
    _ln_body(x_ref, g_ref, b_ref, o_ref)


def _tc_layernorm_into(acc, rows, gamma, beta, row0, bs):
    n, d = rows.shape
    block0 = row0 // TC_TW
    row_spec = pl.BlockSpec((TC_TW, d), lambda i: (i, 0))
    vec_spec = pl.BlockSpec((1, d), lambda i: (0, 0))
    in_specs = [row_spec, vec_spec, vec_spec]
    operands = [rows, gamma.reshape(1, d), beta.reshape(1, d)]
    body = _ln_body
    aliases = {}
    if acc is not None:
        in_specs.insert(0, pl.BlockSpec(memory_space=pl.ANY))
        operands.insert(0, acc)
        aliases = {0: 0}
        body = _ln_body_aliased
    return pl.pallas_call(
        body,
        grid=(n // TC_TW,),
        in_specs=in_specs,
        out_specs=pl.BlockSpec((TC_TW, d), lambda i: (block0 + i, 0)),
        out_shape=jax.ShapeDtypeStruct((bs, d), jnp.float32),
        input_output_aliases=aliases,
        compiler_params=pltpu.CompilerParams(
            dimension_semantics=("parallel",)),
    )(*operands)


NCHUNK = 1


def kernel(input_ids, position_ids, word_table, pos_table, gamma, beta):
    b, s = input_ids.shape
    d = word_table.shape[1]
    bs = b * s
    ids = input_ids.reshape(-1)
    pids = position_ids.reshape(-1)

    nc = bs // NCHUNK
    summed = [
        _sc_gather_sum(word_table, pos_table,
                       lax.slice(ids, (k * nc,), ((k + 1) * nc,)),
                       lax.slice(pids, (k * nc,), ((k + 1) * nc,)))
        for k in range(NCHUNK)
    ]
    acc = None
    for k, rows_k in enumerate(summed):
        acc = _tc_layernorm_into(acc, rows_k, gamma, beta, k * nc, bs)
    return acc.reshape(b, s, d)
